# baseline (speedup 1.0000x reference)
import functools

import numpy as np

import jax
import jax.numpy as jnp
from jax import lax
from jax.experimental import pallas as pl
from jax.experimental.pallas import tpu as pltpu
from jax.experimental.pallas import tpu_sc as plsc

_N = 50000
_E = 800000
_G = 64
_NP = 50048
_RPT = _NP // 16
_NQ = _NP // 4
_NQR = _N // 4

_K = 3
_CHUNK = _K * 128
_CPT = 132
_EPT = _CPT * _CHUNK
_EP = 16 * _EPT
_IROWS_PT = _CPT * _K

_PAD_DST = (_N + (np.arange(_EP - _E) % (_NP - _N))).astype(np.int32)
_BP = 3128
_NBP = _NQ // _BP
_EPS = 1e-5


def _sc_agg(xt, src2, dst2):
    mesh = plsc.VectorSubcoreMesh(core_axis_name="c", subcore_axis_name="s")

    @functools.partial(
        pl.kernel,
        out_type=jax.ShapeDtypeStruct((2, _NP, 32), jnp.float32),
        mesh=mesh,
        scratch_types=[
            pltpu.VMEM((2, _CHUNK), jnp.int32),
            pltpu.VMEM((2, _CHUNK), jnp.int32),
            pltpu.VMEM((2, _CHUNK, 32), jnp.float32),
            pltpu.VMEM_SHARED((_NP, 32), jnp.float32),
            pltpu.SemaphoreType.DMA,
            pltpu.SemaphoreType.DMA,
            pltpu.SemaphoreType.DMA,
            pltpu.SemaphoreType.DMA,
            pltpu.SemaphoreType.DMA,
            pltpu.SemaphoreType.DMA,
        ],
        compiler_params=pltpu.CompilerParams(use_tc_tiling_on_sc=False),
    )
    def agg(xt_hbm, src_hbm, dst_hbm, out_hbm, sidx, didx, rows, acc,
            g0, g1, i0, i1, s0, s1):
        c = lax.axis_index("c")
        s = lax.axis_index("s")
        gsems = (g0, g1)
        isems = (i0, i1)
        ssems = (s0, s1)

        row0 = s * _RPT
        pltpu.sync_copy(xt_hbm.at[c, pl.ds(row0, _RPT)],
                        acc.at[pl.ds(row0, _RPT)])
        plsc.subcore_barrier()

        ebase = s * _IROWS_PT

        def idx_descs(k, buf):
            e0 = ebase * 128 + k * _CHUNK
            return (pltpu.make_async_copy(src_hbm.at[pl.ds(e0, _CHUNK)],
                                          sidx.at[buf], isems[buf]),
                    pltpu.make_async_copy(dst_hbm.at[pl.ds(e0, _CHUNK)],
                                          didx.at[buf], isems[buf]))

        def fire_gathers(buf):
            pltpu.async_copy(xt_hbm.at[c].at[sidx.at[buf]],
                             rows.at[buf], gsems[buf])

        def fire_scatters(buf):
            pltpu.async_copy(rows.at[buf], acc.at[didx.at[buf]],
                             ssems[buf], add=True)

        def drain(sem, buf):
            pltpu.make_async_copy(xt_hbm.at[0, pl.ds(0, _CHUNK)],
                                  rows.at[buf], sem).wait()

        da, db = idx_descs(0, 0)
        da.start()
        db.start()
        da.wait()
        db.wait()
        fire_gathers(0)

        def outer(i, carry):
            for b in range(2):
                k = i * 2 + b
                nb = 1 - b

                @pl.when(k >= 1)
                def _():
                    drain(ssems[nb], nb)

                @pl.when(k < _CPT - 1)
                def _():
                    d1, d2 = idx_descs(k + 1, nb)
                    d1.start()
                    d2.start()

                drain(gsems[b], b)
                fire_scatters(b)

                @pl.when(k < _CPT - 1)
                def _():
                    d1, d2 = idx_descs(k + 1, nb)
                    d1.wait()
                    d2.wait()
                    fire_gathers(nb)
            return carry

        lax.fori_loop(0, _CPT // 2, outer, 0)
        drain(ssems[1], 1)
        plsc.subcore_barrier()

        pltpu.sync_copy(acc.at[pl.ds(row0, _RPT)],
                        out_hbm.at[c, pl.ds(row0, _RPT)])

    return agg(xt, src2, dst2)


def _tc_mlp_stats(hp, W1, b1, W2, b2):
    do = W2.shape[1]
    do4 = 4 * do

    def body(h0_ref, h1_ref, w1_ref, b1_ref, w2_ref, b2_ref,
             y_ref, st_ref, sacc):
        i = pl.program_id(0)
        w1 = w1_ref[...]
        w2 = w2_ref[...]
        ys = []
        for r in range(4):
            h_r = jnp.concatenate([h0_ref[:, 32 * r:32 * r + 32],
                                   h1_ref[:, 32 * r:32 * r + 32]], axis=1)
            t_r = jnp.maximum(
                jnp.dot(h_r, w1, preferred_element_type=jnp.float32)
                + b1_ref[0, :], 0.0)
            ys.append(jnp.dot(t_r, w2, preferred_element_type=jnp.float32)
                      + b2_ref[0, :])
        y = jnp.concatenate(ys, axis=1)
        y_ref[...] = y
        rid = i * _BP + lax.broadcasted_iota(jnp.int32, (_BP, 1), 0)
        ym = jnp.where(rid < _NQR, y, 0.0)
        s1_4 = jnp.sum(ym, axis=0)
        s2_4 = jnp.sum(ym * ym, axis=0)
        s1 = (s1_4[0:do] + s1_4[do:2 * do]
              + s1_4[2 * do:3 * do] + s1_4[3 * do:4 * do])
        s2 = (s2_4[0:do] + s2_4[do:2 * do]
              + s2_4[2 * do:3 * do] + s2_4[3 * do:4 * do])
        upd = jnp.concatenate([s1[None, :], s2[None, :]], axis=0)
        prev = jnp.where(i == 0, jnp.zeros_like(upd), sacc[...])
        sacc[...] = prev + upd

        @pl.when(i == _NBP - 1)
        def _():
            st_ref[...] = sacc[...]

    return pl.pallas_call(
        body,
        grid=(_NBP,),
        in_specs=[
            pl.BlockSpec((_BP, 128), lambda i: (i, 0)),
            pl.BlockSpec((_BP, 128), lambda i: (i + _NBP, 0)),
            pl.BlockSpec((64, 64), lambda i: (0, 0)),
            pl.BlockSpec((1, 64), lambda i: (0, 0)),
            pl.BlockSpec((64, do), lambda i: (0, 0)),
            pl.BlockSpec((1, do), lambda i: (0, 0)),
        ],
        out_specs=[
            pl.BlockSpec((_BP, do4), lambda i: (i, 0)),
            pl.BlockSpec((2, do), lambda i: (0, 0)),
        ],
        out_shape=[
            jax.ShapeDtypeStruct((_NQ, do4), jnp.float32),
            jax.ShapeDtypeStruct((2, do), jnp.float32),
        ],
        scratch_shapes=[pltpu.VMEM((2, do), jnp.float32)],
        compiler_params=pltpu.CompilerParams(
            dimension_semantics=("arbitrary",)),
    )(hp, hp, W1, b1.reshape(1, -1), W2, b2.reshape(1, -1))


def _bn_coeffs(st_ref, g_ref, be_ref, reps):
    mu = st_ref[0, :] * (1.0 / _N)
    var = st_ref[1, :] * (1.0 / _N) - mu * mu
    scale = g_ref[0, :] * lax.rsqrt(var + _EPS)
    shift = be_ref[0, :] - mu * scale
    return (jnp.concatenate([scale] * reps),
            jnp.concatenate([shift] * reps))


def _tc_norm(yp, st, g, be):
    def body(y_ref, st_ref, g_ref, be_ref, o_ref):
        scale4, shift4 = _bn_coeffs(st_ref, g_ref, be_ref, 4)
        yn = jnp.maximum(y_ref[...] * scale4[None, :] + shift4[None, :], 0.0)
        for cc in range(2):
            o_ref[cc] = jnp.concatenate(
                [yn[:, 64 * r + 32 * cc: 64 * r + 32 * cc + 32]
                 for r in range(4)], axis=1)

    return pl.pallas_call(
        body,
        grid=(_NBP,),
        in_specs=[
            pl.BlockSpec((_BP, 256), lambda i: (i, 0)),
            pl.BlockSpec((2, 64), lambda i: (0, 0)),
            pl.BlockSpec((1, 64), lambda i: (0, 0)),
            pl.BlockSpec((1, 64), lambda i: (0, 0)),
        ],
        out_specs=pl.BlockSpec((2, _BP, 128), lambda i: (0, i, 0)),
        out_shape=jax.ShapeDtypeStruct((2, _NQ, 128), jnp.float32),
        compiler_params=pltpu.CompilerParams(
            dimension_semantics=("arbitrary",)),
    )(yp, st, g.reshape(1, -1), be.reshape(1, -1))


def _tc_norm_pool(yp, st, g, be, batchT):
    def body(y_ref, st_ref, g_ref, be_ref, b_ref, o_ref, pacc, cacc):
        i = pl.program_id(0)
        scale4, shift4 = _bn_coeffs(st_ref, g_ref, be_ref, 4)
        yn = jnp.maximum(y_ref[...] * scale4[None, :] + shift4[None, :], 0.0)
        gi = lax.broadcasted_iota(jnp.int32, (_G, _BP), 0)
        ps = jnp.zeros((_G, 32), jnp.float32)
        cs = jnp.zeros((_G, 1), jnp.float32)
        for r in range(4):
            oh = (b_ref[0, r:r + 1] == gi).astype(jnp.float32)
            ps = ps + jnp.dot(oh, yn[:, 32 * r:32 * r + 32],
                              preferred_element_type=jnp.float32)
            cs = cs + jnp.sum(oh, axis=1, keepdims=True)
        pprev = jnp.where(i == 0, jnp.zeros_like(ps), pacc[...])
        cprev = jnp.where(i == 0, jnp.zeros_like(cs), cacc[...])
        pacc[...] = pprev + ps
        cacc[...] = cprev + cs

        @pl.when(i == _NBP - 1)
        def _():
            o_ref[...] = pacc[...] / jnp.maximum(cacc[...], 1.0)

    return pl.pallas_call(
        body,
        grid=(_NBP,),
        in_specs=[
            pl.BlockSpec((_BP, 128), lambda i: (i, 0)),
            pl.BlockSpec((2, 32), lambda i: (0, 0)),
            pl.BlockSpec((1, 32), lambda i: (0, 0)),
            pl.BlockSpec((1, 32), lambda i: (0, 0)),
            pl.BlockSpec((1, 4, _BP), lambda i: (i, 0, 0)),
        ],
        out_specs=pl.BlockSpec((_G, 32), lambda i: (0, 0)),
        out_shape=jax.ShapeDtypeStruct((_G, 32), jnp.float32),
        scratch_shapes=[
            pltpu.VMEM((_G, 32), jnp.float32),
            pltpu.VMEM((_G, 1), jnp.float32),
        ],
        compiler_params=pltpu.CompilerParams(
            dimension_semantics=("arbitrary",)),
    )(yp, st, g.reshape(1, -1), be.reshape(1, -1), batchT)


def kernel(x, edge_index, batch,
           W1_0, b1_0, W2_0, b2_0, g_0, be_0,
           W1_1, b1_1, W2_1, b2_1, g_1, be_1,
           W1_2, b1_2, W2_2, b2_2, g_2, be_2):
    params = [(W1_0, b1_0, W2_0, b2_0, g_0, be_0),
              (W1_1, b1_1, W2_1, b2_1, g_1, be_1),
              (W1_2, b1_2, W2_2, b2_2, g_2, be_2)]

    npad = _EP - _E
    src_p = jnp.concatenate([edge_index[0],
                             jnp.zeros((npad,), jnp.int32)])
    dst_p = jnp.concatenate([edge_index[1], jnp.asarray(_PAD_DST)])
    src2 = src_p
    dst2 = dst_p
    batchT = jnp.pad(batch, (0, _NP - _N),
                     constant_values=_G).reshape(_NBP, _BP, 4).transpose(0, 2, 1)

    xtp = (jnp.pad(x, ((0, _NP - _N), (0, 0)))
           .reshape(_NQ, 4, 2, 32).transpose(2, 0, 1, 3)
           .reshape(2 * _NQ, 128))

    out = None
    for l in range(3):
        W1, b1, W2, b2, g, be = params[l]
        hh = _sc_agg(xtp.reshape(2, _NP, 32), src2, dst2)
        hp = hh.reshape(2 * _NQ, 128)
        yp, st = _tc_mlp_stats(hp, W1, b1, W2, b2)
        if l < 2:
            xtp = _tc_norm(yp, st, g, be).reshape(2 * _NQ, 128)
        else:
            out = _tc_norm_pool(yp, st, g, be, batchT)
    return out
